# baseline (speedup 1.0000x reference)
import functools

import jax
import jax.numpy as jnp
from jax import lax
from jax.experimental import pallas as pl
from jax.experimental.pallas import tpu as pltpu
from jax.experimental.pallas import tpu_sc as plsc

NUSER = 1000000
NITEM = 100000
HID = 64
BATCH = 4096

_info = plsc.get_sparse_core_info()
_NC, _NS, _NL = _info.num_cores, _info.num_subcores, _info.num_lanes
_NW = _NC * _NS
_BPW = BATCH // _NW
_UNBUF = 8
_INBUF = 4
_NG = _BPW // 16


@functools.partial(
    pl.kernel,
    mesh=plsc.VectorSubcoreMesh(core_axis_name="c", subcore_axis_name="s"),
    out_type=[
        jax.ShapeDtypeStruct((HID, BATCH), jnp.float32),
        jax.ShapeDtypeStruct((HID, BATCH), jnp.float32),
    ],
    scratch_types=(
        [pltpu.VMEM((_BPW,), jnp.int32)] * 2
        + [pltpu.VMEM((HID, 128), jnp.float32)] * (_UNBUF + _INBUF)
        + [pltpu.VMEM((HID, _BPW), jnp.float32)] * 2
        + [pltpu.SemaphoreType.DMA] * 2
    ),
    compiler_params=pltpu.CompilerParams(
        needs_layout_passes=False, disable_bounds_checks=True
    ),
)
def _lookup(user_hbm, item_hbm, uft_hbm, ift_hbm, uout_hbm, iout_hbm,
            uidx_v, iidx_v,
            ublk0, ublk1, ublk2, ublk3, ublk4, ublk5, ublk6, ublk7,
            iblk0, iblk1, iblk2, iblk3,
            ucols_v, icols_v, usem, isem):
    ublks = (ublk0, ublk1, ublk2, ublk3, ublk4, ublk5, ublk6, ublk7)
    iblks = (iblk0, iblk1, iblk2, iblk3)
    wid = lax.axis_index("s") * _NC + lax.axis_index("c")
    base = pl.multiple_of(wid * _BPW, _BPW)
    pltpu.sync_copy(user_hbm.at[pl.ds(base, _BPW)], uidx_v)
    pltpu.sync_copy(item_hbm.at[pl.ds(base, _BPW)], iidx_v)
    lanes = lax.iota(jnp.int32, _NL)

    def extract(blk, cols, rr, r):
        rr_v = jnp.full((_NL,), rr, jnp.int32)
        r_v = jnp.full((_NL,), r, jnp.int32)
        for k in range(HID // _NL):
            cvec = k * _NL + lanes
            val = plsc.load_gather(blk, [cvec, rr_v])
            plsc.store_scatter(cols, [cvec, r_v], val)

    def fire(tbl, blk, sem, vec, h):
        r0 = pl.multiple_of((vec[h] >> 7) * 128, 128)
        pltpu.async_copy(tbl.at[:, pl.ds(r0, 128)], blk, sem)

    uvec0 = uidx_v[pl.ds(0, 16)]
    ivec0 = iidx_v[pl.ds(0, 16)]
    for b in range(_UNBUF):
        fire(uft_hbm, ublks[b], usem, uvec0, b)
    for b in range(_INBUF):
        fire(ift_hbm, iblks[b], isem, ivec0, b)

    def group(g):
        uvec = uidx_v[pl.ds(g * 16, 16)]
        ivec = iidx_v[pl.ds(g * 16, 16)]
        gn = jnp.minimum(g + 1, _NG - 1) * 16
        unxt = uidx_v[pl.ds(gn, 16)]
        inxt = iidx_v[pl.ds(gn, 16)]
        not_last = g < _NG - 1
        for h in range(16):
            ub = h % _UNBUF
            ib = h % _INBUF
            r = g * 16 + h
            pltpu.make_async_copy(
                uft_hbm.at[:, pl.ds(0, 128)], ublks[ub], usem).wait()
            extract(ublks[ub], ucols_v, uvec[h] & 127, r)
            if h < 16 - _UNBUF:
                fire(uft_hbm, ublks[ub], usem, uvec, h + _UNBUF)
            else:
                @pl.when(not_last)
                def _():
                    fire(uft_hbm, ublks[ub], usem, unxt, h + _UNBUF - 16)
            pltpu.make_async_copy(
                ift_hbm.at[:, pl.ds(0, 128)], iblks[ib], isem).wait()
            extract(iblks[ib], icols_v, ivec[h] & 127, r)
            if h < 16 - _INBUF:
                fire(ift_hbm, iblks[ib], isem, ivec, h + _INBUF)
            else:
                @pl.when(not_last)
                def _():
                    fire(ift_hbm, iblks[ib], isem, inxt, h + _INBUF - 16)

    pl.loop(0, _NG)(group)
    uw = pltpu.async_copy(ucols_v, uout_hbm.at[:, pl.ds(base, _BPW)], usem)
    iw = pltpu.async_copy(icols_v, iout_hbm.at[:, pl.ds(base, _BPW)], isem)
    uw.wait()
    iw.wait()


def kernel(user, item, user_factor, item_factor):
    user = user.astype(jnp.int32)
    item = item.astype(jnp.int32)
    uout_t, iout_t = _lookup(user, item, user_factor.T, item_factor.T)
    return (uout_t.T, iout_t.T)


# speedup vs baseline: 1.0099x; 1.0099x over previous
import functools

import jax
import jax.numpy as jnp
from jax import lax
from jax.experimental import pallas as pl
from jax.experimental.pallas import tpu as pltpu
from jax.experimental.pallas import tpu_sc as plsc

NUSER = 1000000
NITEM = 100000
HID = 64
BATCH = 4096

_info = plsc.get_sparse_core_info()
_NC, _NS, _NL = _info.num_cores, _info.num_subcores, _info.num_lanes
_NW = _NC * _NS
_BPW = BATCH // _NW
_NBUF = 4
_NG = _BPW // 16


@functools.partial(
    pl.kernel,
    mesh=plsc.VectorSubcoreMesh(core_axis_name="c", subcore_axis_name="s"),
    out_type=[
        jax.ShapeDtypeStruct((HID, BATCH), jnp.float32),
        jax.ShapeDtypeStruct((HID, BATCH), jnp.float32),
    ],
    scratch_types=(
        [pltpu.VMEM((_BPW,), jnp.int32)] * 2
        + [pltpu.VMEM((HID, 128), jnp.float32)] * (2 * _NBUF)
        + [pltpu.VMEM((HID, _BPW), jnp.float32)] * 2
        + [pltpu.SemaphoreType.DMA] * 2
    ),
    compiler_params=pltpu.CompilerParams(
        needs_layout_passes=False, disable_bounds_checks=True
    ),
)
def _lookup(user_hbm, item_hbm, uft_hbm, ift_hbm, uout_hbm, iout_hbm,
            uidx_v, iidx_v,
            ublk0, ublk1, ublk2, ublk3, iblk0, iblk1, iblk2, iblk3,
            ucols_v, icols_v, usem, isem):
    ublks = (ublk0, ublk1, ublk2, ublk3)
    iblks = (iblk0, iblk1, iblk2, iblk3)
    wid = lax.axis_index("s") * _NC + lax.axis_index("c")
    base = pl.multiple_of(wid * _BPW, _BPW)
    pltpu.sync_copy(user_hbm.at[pl.ds(base, _BPW)], uidx_v)
    pltpu.sync_copy(item_hbm.at[pl.ds(base, _BPW)], iidx_v)
    lanes = lax.iota(jnp.int32, _NL)

    def extract(blk, cols, rr, r):
        rr_v = jnp.full((_NL,), rr, jnp.int32)
        r_v = jnp.full((_NL,), r, jnp.int32)
        for k in range(HID // _NL):
            cvec = k * _NL + lanes
            val = plsc.load_gather(blk, [cvec, rr_v])
            plsc.store_scatter(cols, [cvec, r_v], val)

    def fire(tbl, blk, sem, vec, h):
        r0 = pl.multiple_of((vec[h] >> 7) * 128, 128)
        pltpu.async_copy(tbl.at[:, pl.ds(r0, 128)], blk, sem)

    uvec0 = uidx_v[pl.ds(0, 16)]
    ivec0 = iidx_v[pl.ds(0, 16)]
    for b in range(_NBUF):
        fire(uft_hbm, ublks[b], usem, uvec0, b)
        fire(ift_hbm, iblks[b], isem, ivec0, b)

    def group(g):
        uvec = uidx_v[pl.ds(g * 16, 16)]
        ivec = iidx_v[pl.ds(g * 16, 16)]
        gn = jnp.minimum(g + 1, _NG - 1) * 16
        unxt = uidx_v[pl.ds(gn, 16)]
        inxt = iidx_v[pl.ds(gn, 16)]
        not_last = g < _NG - 1
        for h in range(16):
            b = h % _NBUF
            r = g * 16 + h
            pltpu.make_async_copy(
                uft_hbm.at[:, pl.ds(0, 128)], ublks[b], usem).wait()
            extract(ublks[b], ucols_v, uvec[h] & 127, r)
            if h < 16 - _NBUF:
                fire(uft_hbm, ublks[b], usem, uvec, h + _NBUF)
            else:
                @pl.when(not_last)
                def _():
                    fire(uft_hbm, ublks[b], usem, unxt, h + _NBUF - 16)
            pltpu.make_async_copy(
                ift_hbm.at[:, pl.ds(0, 128)], iblks[b], isem).wait()
            extract(iblks[b], icols_v, ivec[h] & 127, r)
            if h < 16 - _NBUF:
                fire(ift_hbm, iblks[b], isem, ivec, h + _NBUF)
            else:
                @pl.when(not_last)
                def _():
                    fire(ift_hbm, iblks[b], isem, inxt, h + _NBUF - 16)

    pl.loop(0, _NG)(group)
    uw = pltpu.async_copy(ucols_v, uout_hbm.at[:, pl.ds(base, _BPW)], usem)
    iw = pltpu.async_copy(icols_v, iout_hbm.at[:, pl.ds(base, _BPW)], isem)
    uw.wait()
    iw.wait()


def kernel(user, item, user_factor, item_factor):
    user = user.astype(jnp.int32)
    item = item.astype(jnp.int32)
    uout_t, iout_t = _lookup(user, item, user_factor.T, item_factor.T)
    return (uout_t.T, iout_t.T)
